# baseline (speedup 1.0000x reference)
import functools

import jax
import jax.numpy as jnp
import numpy as np
from jax import lax
from jax.experimental import pallas as pl
from jax.experimental.pallas import tpu as pltpu
from jax.experimental.pallas import tpu_sc as plsc

_B = 4
_L = 16
_T = 16384
_N = 65536
_RES = [int(np.floor(16.0 * np.exp(l * (np.log(256.0) - np.log(16.0)) / 15.0)))
        for l in range(_L)]
_HASH_K = int(np.uint32(2654435761).view(np.int32))
_CHUNK = 4096


def _sc_feats_body(xt_hbm, out_hbm, tb0, tb1, tb2, tb3, obuf, ixx, wxx, uxx):
    wid = lax.axis_index("s") * 2 + lax.axis_index("c")
    lvl = wid >> 1
    comp = wid & 1

    r = jnp.float32(0.0)
    for ll in range(_L):
        r = jnp.where(lvl == ll, jnp.float32(_RES[ll]), r)
    ri = r.astype(jnp.int32)

    pltpu.sync_copy(xt_hbm.at[comp, 0, lvl], tb0)
    pltpu.sync_copy(xt_hbm.at[comp, 1, lvl], tb1)
    pltpu.sync_copy(xt_hbm.at[comp, 2, lvl], tb2)
    pltpu.sync_copy(xt_hbm.at[comp, 3, lvl], tb3)

    lane2 = lax.iota(jnp.int32, 16) * 2 + 1
    inv512 = jnp.float32(1.0 / 512.0)
    one = jnp.float32(1.0)
    K = jnp.int32(_HASH_K)

    for gx in range(16):
        tx = (lane2 + gx * 32) * ri
        ix0 = tx >> 9
        wx = tx.astype(jnp.float32) * inv512 - ix0.astype(jnp.float32)
        ixx[pl.ds(gx * 16, 16)] = ix0
        wxx[pl.ds(gx * 16, 16)] = wx
        uxx[pl.ds(gx * 16, 16)] = one - wx

    def chunk_body(ch, carry):
        def row_body(yy, carry2):
            y = ch * 16 + yy
            ty = (2 * y + 1) * ri
            iy0 = ty >> 9
            wy = ty.astype(jnp.float32) * inv512 - iy0.astype(jnp.float32)
            vy = one - wy
            a0 = iy0 * K
            a1 = (iy0 + 1) * K
            for gx in range(16):
                ix0 = ixx[pl.ds(gx * 16, 16)]
                wx = wxx[pl.ds(gx * 16, 16)]
                ux = uxx[pl.ds(gx * 16, 16)]
                ix1 = ix0 + 1
                h00 = (ix0 ^ a0) & (_T - 1)
                h10 = (ix1 ^ a0) & (_T - 1)
                h01 = (ix0 ^ a1) & (_T - 1)
                h11 = (ix1 ^ a1) & (_T - 1)
                w00 = ux * vy
                w10 = wx * vy
                w01 = ux * wy
                w11 = wx * wy
                off = yy * 256 + gx * 16
                for row0, tb in enumerate((tb0, tb1, tb2, tb3)):
                    f = (plsc.load_gather(tb, [h00]) * w00
                         + plsc.load_gather(tb, [h10]) * w10
                         + plsc.load_gather(tb, [h01]) * w01
                         + plsc.load_gather(tb, [h11]) * w11)
                    obuf[row0, pl.ds(off, 16)] = f
            return carry2

        lax.fori_loop(0, 16, row_body, 0)
        n0 = ch * _CHUNK
        row = 2 * lvl + comp
        pltpu.sync_copy(obuf.at[0], out_hbm.at[0, row, pl.ds(n0, _CHUNK)])
        pltpu.sync_copy(obuf.at[1], out_hbm.at[1, row, pl.ds(n0, _CHUNK)])
        pltpu.sync_copy(obuf.at[2], out_hbm.at[2, row, pl.ds(n0, _CHUNK)])
        pltpu.sync_copy(obuf.at[3], out_hbm.at[3, row, pl.ds(n0, _CHUNK)])
        return carry

    lax.fori_loop(0, _N // _CHUNK, chunk_body, 0)


_sc_cache = {}


def _get_sc_feats():
    if "k" not in _sc_cache:
        mesh = plsc.VectorSubcoreMesh(core_axis_name="c", subcore_axis_name="s")
        _sc_cache["k"] = pl.kernel(
            _sc_feats_body,
            out_type=jax.ShapeDtypeStruct((_B, 2 * _L, _N), jnp.float32),
            mesh=mesh,
            scratch_types=[
                pltpu.VMEM((_T,), jnp.float32),
                pltpu.VMEM((_T,), jnp.float32),
                pltpu.VMEM((_T,), jnp.float32),
                pltpu.VMEM((_T,), jnp.float32),
                pltpu.VMEM((4, _CHUNK), jnp.float32),
                pltpu.VMEM((256,), jnp.int32),
                pltpu.VMEM((256,), jnp.float32),
                pltpu.VMEM((256,), jnp.float32),
            ],
            compiler_params=pltpu.CompilerParams(needs_layout_passes=False),
        )
    return _sc_cache["k"]


_NT = 4096


def _style(s, Aw, Ab):
    return lax.dot_general(s, Aw, (((1,), (1,)), ((), ())),
                           preferred_element_type=jnp.float32) + Ab


def _modw(W, style):
    w = W[None, :, :] * style[:, None, :]
    d = lax.rsqrt(jnp.sum(w * w, axis=2, keepdims=True) + 1e-8)
    return w * d


def _mlp_body(s_ref, W0_ref, b0_ref, A0w_ref, A0b_ref,
              W1_ref, b1_ref, A1w_ref, A1b_ref,
              W2_ref, b2_ref, A2w_ref, A2b_ref, f_ref, o_ref,
              W0s, W1s, W2s):
    @pl.when(pl.program_id(0) == 0)
    def _init():
        W0s[...] = jnp.zeros((128, 128), jnp.float32)
        W1s[...] = jnp.zeros((128, 128), jnp.float32)
        W2s[...] = jnp.zeros((32, 128), jnp.float32)
        s = s_ref[...]
        w0 = _modw(W0_ref[...], _style(s, A0w_ref[...], A0b_ref[...]))
        w1 = _modw(W1_ref[...], _style(s, A1w_ref[...], A1b_ref[...]))
        w2 = _modw(W2_ref[...], _style(s, A2w_ref[...], A2b_ref[...]))
        for b in range(_B):
            W0s[pl.ds(32 * b, 32), pl.ds(32 * b, 32)] = w0[b]
            W1s[pl.ds(32 * b, 32), pl.ds(32 * b, 32)] = w1[b]
            W2s[pl.ds(8 * b, 3), pl.ds(32 * b, 32)] = w2[b]

    f = f_ref[...].reshape(128, _NT)
    h = jnp.maximum(jnp.dot(W0s[...], f, preferred_element_type=jnp.float32)
                    + b0_ref[...], 0.0)
    h = jnp.maximum(jnp.dot(W1s[...], h, preferred_element_type=jnp.float32)
                    + b1_ref[...], 0.0)
    o = jnp.tanh(jnp.dot(W2s[...], h, preferred_element_type=jnp.float32)
                 + b2_ref[...])
    o_ref[...] = o


def _full(shape):
    return pl.BlockSpec(shape, lambda n: tuple(0 for _ in shape))


def kernel(x, s, W0, b0, A0w, A0b, W1, b1, A1w, A1b, W2, b2, A2w, A2b):
    xt = x.reshape(_B, _L, _T, 2).transpose(3, 0, 1, 2)
    feats = _get_sc_feats()(xt)

    b0c = jnp.tile(b0, _B).reshape(128, 1)
    b1c = jnp.tile(b1, _B).reshape(128, 1)
    b2c = jnp.tile(jnp.concatenate([b2, jnp.zeros(5, jnp.float32)]),
                   _B).reshape(32, 1)

    out = pl.pallas_call(
        _mlp_body,
        grid=(_N // _NT,),
        in_specs=[
            _full((_B, 512)),
            _full((32, 32)), _full((128, 1)), _full((32, 512)), _full((1, 32)),
            _full((32, 32)), _full((128, 1)), _full((32, 512)), _full((1, 32)),
            _full((3, 32)), _full((32, 1)), _full((32, 512)), _full((1, 32)),
            pl.BlockSpec((_B, 32, _NT), lambda n: (0, 0, n)),
        ],
        out_specs=pl.BlockSpec((32, _NT), lambda n: (0, n)),
        out_shape=jax.ShapeDtypeStruct((32, _N), jnp.float32),
        scratch_shapes=[
            pltpu.VMEM((128, 128), jnp.float32),
            pltpu.VMEM((128, 128), jnp.float32),
            pltpu.VMEM((32, 128), jnp.float32),
        ],
        compiler_params=pltpu.CompilerParams(
            dimension_semantics=("arbitrary",)),
    )(s, W0, b0c, A0w, A0b.reshape(1, 32),
      W1, b1c, A1w, A1b.reshape(1, 32),
      W2, b2c, A2w, A2b.reshape(1, 32),
      feats)
    return out.reshape(_B, 8, 256, 256)[:, :3]


# speedup vs baseline: 1.7184x; 1.7184x over previous
import functools

import jax
import jax.numpy as jnp
import numpy as np
from jax import lax
from jax.experimental import pallas as pl
from jax.experimental.pallas import tpu as pltpu
from jax.experimental.pallas import tpu_sc as plsc

_B = 4
_L = 16
_T = 16384
_N = 65536
_RES = [int(np.floor(16.0 * np.exp(l * (np.log(256.0) - np.log(16.0)) / 15.0)))
        for l in range(_L)]
_HASH_K = int(np.uint32(2654435761).view(np.int32))
_CHUNK = 4096


def _sc_feats_body(x_hbm, out_hbm, tb0, tb1, obuf):
    wid = lax.axis_index("s") * 2 + lax.axis_index("c")
    lvl = wid >> 1
    pair = wid & 1
    b0 = 2 * pair
    b1 = b0 + 1

    r = jnp.float32(0.0)
    for ll in range(_L):
        r = jnp.where(lvl == ll, jnp.float32(_RES[ll]), r)

    pltpu.sync_copy(x_hbm.at[b0, lvl], tb0)
    pltpu.sync_copy(x_hbm.at[b1, lvl], tb1)

    ri = r.astype(jnp.int32)
    lane2 = lax.iota(jnp.int32, 16) * 2 + 1
    inv512 = jnp.float32(1.0 / 512.0)
    one = jnp.float32(1.0)
    K = jnp.int32(_HASH_K)

    def chunk_body(ch, carry):
        def row_body(yy, carry2):
            y = ch * 16 + yy
            ty = (2 * y + 1) * ri
            iy0 = ty >> 9
            wy = ty.astype(jnp.float32) * inv512 - iy0.astype(jnp.float32)
            vy = one - wy
            a0 = iy0 * K
            a1 = (iy0 + 1) * K
            for gx in range(16):
                tx = (lane2 + gx * 32) * ri
                ix0 = tx >> 9
                wx = tx.astype(jnp.float32) * inv512 - ix0.astype(jnp.float32)
                ux = one - wx
                ix1 = ix0 + 1
                h00 = ((ix0 ^ a0) & (_T - 1)) << 1
                h10 = ((ix1 ^ a0) & (_T - 1)) << 1
                h01 = ((ix0 ^ a1) & (_T - 1)) << 1
                h11 = ((ix1 ^ a1) & (_T - 1)) << 1
                g00 = h00 + 1
                g10 = h10 + 1
                g01 = h01 + 1
                g11 = h11 + 1
                w00 = ux * vy
                w10 = wx * vy
                w01 = ux * wy
                w11 = wx * wy
                off = yy * 256 + gx * 16
                for tb, row0 in ((tb0, 0), (tb1, 2)):
                    fx = (plsc.load_gather(tb, [h00]) * w00
                          + plsc.load_gather(tb, [h10]) * w10
                          + plsc.load_gather(tb, [h01]) * w01
                          + plsc.load_gather(tb, [h11]) * w11)
                    fy = (plsc.load_gather(tb, [g00]) * w00
                          + plsc.load_gather(tb, [g10]) * w10
                          + plsc.load_gather(tb, [g01]) * w01
                          + plsc.load_gather(tb, [g11]) * w11)
                    obuf[row0, pl.ds(off, 16)] = fx
                    obuf[row0 + 1, pl.ds(off, 16)] = fy
            return carry2

        lax.fori_loop(0, 16, row_body, 0)
        n0 = ch * _CHUNK
        pltpu.sync_copy(obuf.at[0], out_hbm.at[b0, 2 * lvl, pl.ds(n0, _CHUNK)])
        pltpu.sync_copy(obuf.at[1], out_hbm.at[b0, 2 * lvl + 1, pl.ds(n0, _CHUNK)])
        pltpu.sync_copy(obuf.at[2], out_hbm.at[b1, 2 * lvl, pl.ds(n0, _CHUNK)])
        pltpu.sync_copy(obuf.at[3], out_hbm.at[b1, 2 * lvl + 1, pl.ds(n0, _CHUNK)])
        return carry

    lax.fori_loop(0, _N // _CHUNK, chunk_body, 0)


_sc_cache = {}


def _get_sc_feats():
    if "k" not in _sc_cache:
        mesh = plsc.VectorSubcoreMesh(core_axis_name="c", subcore_axis_name="s")
        _sc_cache["k"] = pl.kernel(
            _sc_feats_body,
            out_type=jax.ShapeDtypeStruct((_B, 2 * _L, _N), jnp.float32),
            mesh=mesh,
            scratch_types=[
                pltpu.VMEM((2 * _T,), jnp.float32),
                pltpu.VMEM((2 * _T,), jnp.float32),
                pltpu.VMEM((4, _CHUNK), jnp.float32),
            ],
            compiler_params=pltpu.CompilerParams(needs_layout_passes=False),
        )
    return _sc_cache["k"]


_NT = 4096


def _style(s, Aw, Ab):
    return lax.dot_general(s, Aw, (((1,), (1,)), ((), ())),
                           preferred_element_type=jnp.float32) + Ab


def _modw(W, style):
    w = W[None, :, :] * style[:, None, :]
    d = lax.rsqrt(jnp.sum(w * w, axis=2, keepdims=True) + 1e-8)
    return w * d


def _mlp_body(s_ref, W0_ref, b0_ref, A0w_ref, A0b_ref,
              W1_ref, b1_ref, A1w_ref, A1b_ref,
              W2_ref, b2_ref, A2w_ref, A2b_ref, f_ref, o_ref,
              W0s, W1s, W2s):
    @pl.when(pl.program_id(0) == 0)
    def _init():
        W0s[...] = jnp.zeros((128, 128), jnp.float32)
        W1s[...] = jnp.zeros((128, 128), jnp.float32)
        W2s[...] = jnp.zeros((32, 128), jnp.float32)
        s = s_ref[...]
        w0 = _modw(W0_ref[...], _style(s, A0w_ref[...], A0b_ref[...]))
        w1 = _modw(W1_ref[...], _style(s, A1w_ref[...], A1b_ref[...]))
        w2 = _modw(W2_ref[...], _style(s, A2w_ref[...], A2b_ref[...]))
        for b in range(_B):
            W0s[pl.ds(32 * b, 32), pl.ds(32 * b, 32)] = w0[b]
            W1s[pl.ds(32 * b, 32), pl.ds(32 * b, 32)] = w1[b]
            W2s[pl.ds(8 * b, 3), pl.ds(32 * b, 32)] = w2[b]

    f = f_ref[...].reshape(128, _NT)
    h = jnp.maximum(jnp.dot(W0s[...], f, preferred_element_type=jnp.float32)
                    + b0_ref[...], 0.0)
    h = jnp.maximum(jnp.dot(W1s[...], h, preferred_element_type=jnp.float32)
                    + b1_ref[...], 0.0)
    o = jnp.tanh(jnp.dot(W2s[...], h, preferred_element_type=jnp.float32)
                 + b2_ref[...])
    o_ref[...] = o


def _full(shape):
    return pl.BlockSpec(shape, lambda n: tuple(0 for _ in shape))


def kernel(x, s, W0, b0, A0w, A0b, W1, b1, A1w, A1b, W2, b2, A2w, A2b):
    feats = _get_sc_feats()(x.reshape(_B, _L, 2 * _T))

    b0c = jnp.tile(b0, _B).reshape(128, 1)
    b1c = jnp.tile(b1, _B).reshape(128, 1)
    b2c = jnp.tile(jnp.concatenate([b2, jnp.zeros(5, jnp.float32)]),
                   _B).reshape(32, 1)

    out = pl.pallas_call(
        _mlp_body,
        grid=(_N // _NT,),
        in_specs=[
            _full((_B, 512)),
            _full((32, 32)), _full((128, 1)), _full((32, 512)), _full((1, 32)),
            _full((32, 32)), _full((128, 1)), _full((32, 512)), _full((1, 32)),
            _full((3, 32)), _full((32, 1)), _full((32, 512)), _full((1, 32)),
            pl.BlockSpec((_B, 32, _NT), lambda n: (0, 0, n)),
        ],
        out_specs=pl.BlockSpec((32, _NT), lambda n: (0, n)),
        out_shape=jax.ShapeDtypeStruct((32, _N), jnp.float32),
        scratch_shapes=[
            pltpu.VMEM((128, 128), jnp.float32),
            pltpu.VMEM((128, 128), jnp.float32),
            pltpu.VMEM((32, 128), jnp.float32),
        ],
        compiler_params=pltpu.CompilerParams(
            dimension_semantics=("arbitrary",)),
    )(s, W0, b0c, A0w, A0b.reshape(1, 32),
      W1, b1c, A1w, A1b.reshape(1, 32),
      W2, b2c, A2w, A2b.reshape(1, 32),
      feats)
    return out.reshape(_B, 8, 256, 256)[:, :3]
